# baseline (speedup 1.0000x reference)
import functools

import jax
import jax.numpy as jnp
from jax import lax
from jax.experimental import pallas as pl
from jax.experimental.pallas import tpu as pltpu
from jax.experimental.pallas import tpu_sc as plsc

_NUM_EXAMPLES = 1024
_SEQ_LEN = 50
_VOCAB = 1000

_LANES = 16
_NVREG = -(-_VOCAB // _LANES)
_PAD = _NVREG * _LANES
_NUM_CORES = 2
_NUM_SUBCORES = 16
_ROWS_PER_W = 2

_LN2 = 0.6931471805599453
_SQRT2 = 1.4142135623730951


def _vlog(x):
    bits = lax.bitcast_convert_type(x, jnp.int32)
    e = lax.shift_right_logical(bits, 23) - 127
    mbits = lax.bitwise_or(lax.bitwise_and(bits, 0x007FFFFF), 0x3F800000)
    m = lax.bitcast_convert_type(mbits, jnp.float32)
    big = m > _SQRT2
    m = jnp.where(big, m * 0.5, m)
    e = jnp.where(big, e + 1, e)
    t = (m - 1.0) / (m + 1.0)
    t2 = t * t
    p = t * (2.0 + t2 * (2.0 / 3.0 + t2 * (0.4 + t2 * (2.0 / 7.0))))
    return e.astype(jnp.float32) * _LN2 + p


def _xlane(x, op):
    dnums = lax.GatherDimensionNumbers(
        offset_dims=(), collapsed_slice_dims=(0,), start_index_map=(0,))
    for step in (1, 2, 4, 8):
        perm = lax.bitwise_xor(lax.iota(jnp.int32, _LANES), step)
        shuf = lax.gather(x, perm.reshape(_LANES, 1), dnums, (1,),
                          mode=lax.GatherScatterMode.PROMISE_IN_BOUNDS)
        x = op(x, shuf)
    return x


def _sc_body(wf_hbm, base_hbm, out_hbm, idx_v, buf):
    wid = lax.axis_index("s") * _NUM_CORES + lax.axis_index("c")
    pltpu.sync_copy(base_hbm, idx_v)
    base = idx_v[...][0]
    row0 = wid * _ROWS_PER_W

    @pl.when(row0 < _SEQ_LEN)
    def _():
        for r in range(_ROWS_PER_W):
            row = row0 + r
            b0 = r * _PAD
            pltpu.sync_copy(wf_hbm.at[pl.ds((base + row) * _VOCAB, _VOCAB)],
                            buf.at[pl.ds(b0, _VOCAB)])
            tail = buf[pl.ds(b0 + _PAD - _LANES, _LANES)]
            lane = lax.iota(jnp.int32, _LANES)
            buf[pl.ds(b0 + _PAD - _LANES, _LANES)] = jnp.where(
                lane < _VOCAB - (_PAD - _LANES), tail, -jnp.inf)

        for r in range(_ROWS_PER_W):
            row = row0 + r
            b0 = r * _PAD
            m = jnp.full((_LANES,), -jnp.inf, jnp.float32)
            for k in range(_NVREG):
                m = jnp.maximum(m, buf[pl.ds(b0 + k * _LANES, _LANES)])
            mb = _xlane(m, jnp.maximum)
            s = jnp.zeros((_LANES,), jnp.float32)
            for k in range(_NVREG):
                s = s + jnp.exp(buf[pl.ds(b0 + k * _LANES, _LANES)] - mb)
            off = mb + _vlog(_xlane(s, jnp.add))
            for k in range(_NVREG):
                buf[pl.ds(b0 + k * _LANES, _LANES)] = (
                    buf[pl.ds(b0 + k * _LANES, _LANES)] - off)
            pltpu.sync_copy(buf.at[pl.ds(b0, _VOCAB)],
                            out_hbm.at[pl.ds(row * _VOCAB, _VOCAB)])


_sc_kernel = functools.partial(
    pl.kernel,
    mesh=plsc.VectorSubcoreMesh(core_axis_name="c", subcore_axis_name="s"),
    out_type=jax.ShapeDtypeStruct((_SEQ_LEN * _VOCAB,), jnp.float32),
    scratch_types=[
        pltpu.VMEM((_LANES,), jnp.int32),
        pltpu.VMEM((_ROWS_PER_W * _PAD,), jnp.float32),
    ],
)(_sc_body)


def kernel(weights, example_idx):
    wf = weights.reshape(_NUM_EXAMPLES * _SEQ_LEN * _VOCAB)
    base = jnp.broadcast_to(
        jnp.asarray(example_idx, jnp.int32) * _SEQ_LEN, (_LANES,))
    return _sc_kernel(wf, base).reshape(_SEQ_LEN, _VOCAB)


# speedup vs baseline: 2.0914x; 2.0914x over previous
import functools

import jax
import jax.numpy as jnp
from jax import lax
from jax.experimental import pallas as pl
from jax.experimental.pallas import tpu as pltpu
from jax.experimental.pallas import tpu_sc as plsc

_NUM_EXAMPLES = 1024
_SEQ_LEN = 50
_VOCAB = 1000

_LANES = 16
_NUM_CORES = 2
_NT = 8
_NR = 13
_KPR = 8
_ROWBLK = 32
_PART = _NR * 2 * _LANES

_LN2 = 0.6931471805599453
_SQRT2 = 1.4142135623730951


def _vlog(x):
    bits = lax.bitcast_convert_type(x, jnp.int32)
    e = lax.shift_right_logical(bits, 23) - 127
    mbits = lax.bitwise_or(lax.bitwise_and(bits, 0x007FFFFF), 0x3F800000)
    m = lax.bitcast_convert_type(mbits, jnp.float32)
    big = m > _SQRT2
    m = jnp.where(big, m * 0.5, m)
    e = jnp.where(big, e + 1, e)
    t = (m - 1.0) / (m + 1.0)
    t2 = t * t
    p = t * (2.0 + t2 * (2.0 / 3.0 + t2 * (0.4 + t2 * (2.0 / 7.0))))
    return e.astype(jnp.float32) * _LN2 + p


def _xlane(x, op):
    dnums = lax.GatherDimensionNumbers(
        offset_dims=(), collapsed_slice_dims=(0,), start_index_map=(0,))
    for step in (1, 2, 4, 8):
        perm = lax.bitwise_xor(lax.iota(jnp.int32, _LANES), step)
        shuf = lax.gather(x, perm.reshape(_LANES, 1), dnums, (1,),
                          mode=lax.GatherScatterMode.PROMISE_IN_BOUNDS)
        x = op(x, shuf)
    return x


def _sc_body(w_hbm, base_hbm, out_hbm, idx_v, scr, part, allp, outb, shared):
    c = lax.axis_index("c")
    s = lax.axis_index("s")
    tc = s % _NT
    rh = s // _NT
    pltpu.sync_copy(base_hbm, idx_v)
    e = idx_v[...][0]

    r0 = c * 24 + rh * (11 + 2 * c)
    rel0 = rh * (11 + 2 * c)
    dma_r0 = pl.multiple_of(c * 24 + rh * 8, 8)
    col0 = pl.multiple_of(tc * 128, 128)

    @pl.when(tc < _NT - 1)
    def _():
        pltpu.sync_copy(w_hbm.at[e, pl.ds(dma_r0, 16), pl.ds(col0, 128)],
                        scr.at[pl.ds(dma_r0, 16), pl.ds(col0, 128)])

        @pl.when(jnp.logical_and(c == 1, rh == 1))
        def _():
            pltpu.sync_copy(w_hbm.at[e, pl.ds(48, 2), pl.ds(col0, 128)],
                            scr.at[pl.ds(48, 2), pl.ds(col0, 128)])

    @pl.when(tc == _NT - 1)
    def _():
        pltpu.sync_copy(w_hbm.at[e, pl.ds(dma_r0, 16), pl.ds(896, 104)],
                        scr.at[pl.ds(dma_r0, 16), pl.ds(896, 104)])

        @pl.when(jnp.logical_and(c == 1, rh == 1))
        def _():
            pltpu.sync_copy(w_hbm.at[e, pl.ds(48, 2), pl.ds(896, 104)],
                            scr.at[pl.ds(48, 2), pl.ds(896, 104)])

        lane = lax.iota(jnp.int32, _LANES)
        ninf = jnp.full((_LANES,), -jnp.inf, jnp.float32)
        for j in range(_NR):
            r = r0 + j
            v = scr[r, pl.ds(col0 + 96, _LANES)]
            scr[r, pl.ds(col0 + 96, _LANES)] = jnp.where(lane < 8, v, ninf)
            scr[r, pl.ds(col0 + 112, _LANES)] = ninf

    for j in range(_NR):
        r = r0 + j
        pm = scr[r, pl.ds(col0, _LANES)]
        for k in range(1, _KPR):
            pm = jnp.maximum(pm, scr[r, pl.ds(col0 + k * _LANES, _LANES)])
        ps = jnp.zeros((_LANES,), jnp.float32)
        for k in range(_KPR):
            ps = ps + jnp.exp(scr[r, pl.ds(col0 + k * _LANES, _LANES)] - pm)
        part[pl.ds(j * 2 * _LANES, _LANES)] = pm
        part[pl.ds((j * 2 + 1) * _LANES, _LANES)] = ps

    pltpu.sync_copy(
        part, shared.at[pl.ds((tc * _ROWBLK + rel0) * 2 * _LANES, _PART)])
    plsc.subcore_barrier()
    for t2 in range(_NT):
        pltpu.sync_copy(
            shared.at[pl.ds((t2 * _ROWBLK + rel0) * 2 * _LANES, _PART)],
            allp.at[pl.ds(t2 * _PART, _PART)])

    for j in range(_NR):
        r = r0 + j
        pm0 = allp[pl.ds(j * 2 * _LANES, _LANES)]
        mx = pm0
        for t2 in range(1, _NT):
            mx = jnp.maximum(
                mx, allp[pl.ds(t2 * _PART + j * 2 * _LANES, _LANES)])
        mx = _xlane(mx, jnp.maximum)
        sm = jnp.zeros((_LANES,), jnp.float32)
        for t2 in range(_NT):
            pmt = allp[pl.ds(t2 * _PART + j * 2 * _LANES, _LANES)]
            pst = allp[pl.ds(t2 * _PART + (j * 2 + 1) * _LANES, _LANES)]
            sm = sm + pst * jnp.exp(pmt - mx)
        off = mx + _vlog(_xlane(sm, jnp.add))
        for k in range(_KPR):
            outb[pl.ds(j * 128 + k * _LANES, _LANES)] = (
                scr[r, pl.ds(col0 + k * _LANES, _LANES)] - off)

    @pl.when(tc < _NT - 1)
    def _():
        for j in range(_NR):
            pltpu.sync_copy(
                outb.at[pl.ds(j * 128, 128)],
                out_hbm.at[pl.ds((r0 + j) * _VOCAB + col0, 128)])

    @pl.when(tc == _NT - 1)
    def _():
        for j in range(_NR):
            pltpu.sync_copy(
                outb.at[pl.ds(j * 128, 104)],
                out_hbm.at[pl.ds((r0 + j) * _VOCAB + 896, 104)])


_sc_kernel = functools.partial(
    pl.kernel,
    mesh=plsc.VectorSubcoreMesh(core_axis_name="c", subcore_axis_name="s"),
    out_type=jax.ShapeDtypeStruct((_SEQ_LEN * _VOCAB,), jnp.float32),
    scratch_types=[
        pltpu.VMEM((_LANES,), jnp.int32),
        pltpu.VMEM((_SEQ_LEN, _VOCAB), jnp.float32),
        pltpu.VMEM((_PART,), jnp.float32),
        pltpu.VMEM((_NT * _PART,), jnp.float32),
        pltpu.VMEM((_NR * 128,), jnp.float32),
        pltpu.VMEM_SHARED((_NT * _ROWBLK * 2 * _LANES,), jnp.float32),
    ],
)(_sc_body)


def kernel(weights, example_idx):
    base = jnp.broadcast_to(jnp.asarray(example_idx, jnp.int32), (_LANES,))
    return _sc_kernel(weights, base).reshape(_SEQ_LEN, _VOCAB)


# speedup vs baseline: 2.0957x; 1.0020x over previous
import functools

import jax
import jax.numpy as jnp
from jax import lax
from jax.experimental import pallas as pl
from jax.experimental.pallas import tpu as pltpu
from jax.experimental.pallas import tpu_sc as plsc

_NUM_EXAMPLES = 1024
_SEQ_LEN = 50
_VOCAB = 1000

_LANES = 16
_NUM_CORES = 2
_NT = 8
_NR = 13
_KPR = 8
_ROWBLK = 32
_PART = _NR * 2 * _LANES

_LN2 = 0.6931471805599453
_SQRT2 = 1.4142135623730951


def _vlog(x):
    bits = lax.bitcast_convert_type(x, jnp.int32)
    e = lax.shift_right_logical(bits, 23) - 127
    mbits = lax.bitwise_or(lax.bitwise_and(bits, 0x007FFFFF), 0x3F800000)
    m = lax.bitcast_convert_type(mbits, jnp.float32)
    big = m > _SQRT2
    m = jnp.where(big, m * 0.5, m)
    e = jnp.where(big, e + 1, e)
    t = (m - 1.0) / (m + 1.0)
    t2 = t * t
    p = t * (2.0 + t2 * (2.0 / 3.0 + t2 * (0.4 + t2 * (2.0 / 7.0))))
    return e.astype(jnp.float32) * _LN2 + p


def _xlane(x, op):
    dnums = lax.GatherDimensionNumbers(
        offset_dims=(), collapsed_slice_dims=(0,), start_index_map=(0,))
    for step in (1, 2, 4, 8):
        perm = lax.bitwise_xor(lax.iota(jnp.int32, _LANES), step)
        shuf = lax.gather(x, perm.reshape(_LANES, 1), dnums, (1,),
                          mode=lax.GatherScatterMode.PROMISE_IN_BOUNDS)
        x = op(x, shuf)
    return x


def _sc_body(w_hbm, base_hbm, out_hbm, idx_v, scr, part, allp, outb, shared):
    c = lax.axis_index("c")
    s = lax.axis_index("s")
    tc = s % _NT
    rh = s // _NT
    pltpu.sync_copy(base_hbm, idx_v)
    e = idx_v[...][0]

    r0 = c * 24 + rh * (11 + 2 * c)
    rel0 = rh * (11 + 2 * c)
    dma_r0 = pl.multiple_of(c * 24 + rh * 8, 8)
    col0 = pl.multiple_of(tc * 128, 128)

    @pl.when(tc < _NT - 1)
    def _():
        pltpu.sync_copy(w_hbm.at[e, pl.ds(dma_r0, 16), pl.ds(col0, 128)],
                        scr.at[pl.ds(dma_r0, 16), pl.ds(col0, 128)])

        @pl.when(jnp.logical_and(c == 1, rh == 1))
        def _():
            pltpu.sync_copy(w_hbm.at[e, pl.ds(48, 2), pl.ds(col0, 128)],
                            scr.at[pl.ds(48, 2), pl.ds(col0, 128)])

    @pl.when(tc == _NT - 1)
    def _():
        pltpu.sync_copy(w_hbm.at[e, pl.ds(dma_r0, 16), pl.ds(896, 104)],
                        scr.at[pl.ds(dma_r0, 16), pl.ds(896, 104)])

        @pl.when(jnp.logical_and(c == 1, rh == 1))
        def _():
            pltpu.sync_copy(w_hbm.at[e, pl.ds(48, 2), pl.ds(896, 104)],
                            scr.at[pl.ds(48, 2), pl.ds(896, 104)])

        lane = lax.iota(jnp.int32, _LANES)
        ninf = jnp.full((_LANES,), -jnp.inf, jnp.float32)
        for j in range(_NR):
            r = r0 + j
            v = scr[r, pl.ds(col0 + 96, _LANES)]
            scr[r, pl.ds(col0 + 96, _LANES)] = jnp.where(lane < 8, v, ninf)
            scr[r, pl.ds(col0 + 112, _LANES)] = ninf

    for j in range(_NR):
        r = r0 + j
        pm = scr[r, pl.ds(col0, _LANES)]
        for k in range(1, _KPR):
            pm = jnp.maximum(pm, scr[r, pl.ds(col0 + k * _LANES, _LANES)])
        ps = jnp.zeros((_LANES,), jnp.float32)
        for k in range(_KPR):
            ps = ps + jnp.exp(scr[r, pl.ds(col0 + k * _LANES, _LANES)] - pm)
        part[pl.ds(j * 2 * _LANES, _LANES)] = pm
        part[pl.ds((j * 2 + 1) * _LANES, _LANES)] = ps

    pltpu.sync_copy(
        part, shared.at[pl.ds((tc * _ROWBLK + rel0) * 2 * _LANES, _PART)])
    plsc.subcore_barrier()
    for t2 in range(_NT):
        pltpu.sync_copy(
            shared.at[pl.ds((t2 * _ROWBLK + rel0) * 2 * _LANES, _PART)],
            allp.at[pl.ds(t2 * _PART, _PART)])

    for j in range(_NR):
        r = r0 + j
        pm0 = allp[pl.ds(j * 2 * _LANES, _LANES)]
        mx = pm0
        for t2 in range(1, _NT):
            mx = jnp.maximum(
                mx, allp[pl.ds(t2 * _PART + j * 2 * _LANES, _LANES)])
        mx = _xlane(mx, jnp.maximum)
        sm = jnp.zeros((_LANES,), jnp.float32)
        for t2 in range(_NT):
            pmt = allp[pl.ds(t2 * _PART + j * 2 * _LANES, _LANES)]
            pst = allp[pl.ds(t2 * _PART + (j * 2 + 1) * _LANES, _LANES)]
            sm = sm + pst * jnp.exp(pmt - mx)
        off = mx + _vlog(_xlane(sm, jnp.add))
        for k in range(_KPR):
            outb[pl.ds(j * 128 + k * _LANES, _LANES)] = (
                scr[r, pl.ds(col0 + k * _LANES, _LANES)] - off)

    @pl.when(tc < _NT - 1)
    def _():
        for j in range(_NR):
            pltpu.sync_copy(
                outb.at[pl.ds(j * 128, 128)],
                out_hbm.at[pl.ds((r0 + j) * _VOCAB + col0, 128)])

    @pl.when(tc == _NT - 1)
    def _():
        for j in range(_NR):
            pltpu.sync_copy(
                outb.at[pl.ds(j * 128, 104)],
                out_hbm.at[pl.ds((r0 + j) * _VOCAB + 896, 104)])


_sc_kernel = functools.partial(
    pl.kernel,
    mesh=plsc.VectorSubcoreMesh(core_axis_name="c", subcore_axis_name="s"),
    out_type=jax.ShapeDtypeStruct((_SEQ_LEN * _VOCAB,), jnp.float32),
    compiler_params=pltpu.CompilerParams(skip_device_barrier=True),
    scratch_types=[
        pltpu.VMEM((_LANES,), jnp.int32),
        pltpu.VMEM((_SEQ_LEN, _VOCAB), jnp.float32),
        pltpu.VMEM((_PART,), jnp.float32),
        pltpu.VMEM((_NT * _PART,), jnp.float32),
        pltpu.VMEM((_NR * 128,), jnp.float32),
        pltpu.VMEM_SHARED((_NT * _ROWBLK * 2 * _LANES,), jnp.float32),
    ],
)(_sc_body)


def kernel(weights, example_idx):
    base = jnp.broadcast_to(jnp.asarray(example_idx, jnp.int32), (_LANES,))
    return _sc_kernel(weights, base).reshape(_SEQ_LEN, _VOCAB)


# speedup vs baseline: 18.1896x; 8.6796x over previous
import functools

import jax
import jax.numpy as jnp
from jax import lax
from jax.experimental import pallas as pl
from jax.experimental.pallas import tpu as pltpu
from jax.experimental.pallas import tpu_sc as plsc

_NUM_EXAMPLES = 1024
_SEQ_LEN = 50
_VOCAB = 1000

_LANES = 16
_NUM_CORES = 2
_ROWS_PER_W = 2
_VPAD = 1024
_NV = _VPAD // _LANES
_CHUNK = 128
_NCH = _VPAD // _CHUNK
_ROW_STRIDE = _SEQ_LEN * _VOCAB * _NUM_EXAMPLES // _SEQ_LEN
_G_STRIDE = 16384

_LN2 = 0.6931471805599453
_SQRT2 = 1.4142135623730951


def _vlog(x):
    bits = lax.bitcast_convert_type(x, jnp.int32)
    e = lax.shift_right_logical(bits, 23) - 127
    mbits = lax.bitwise_or(lax.bitwise_and(bits, 0x007FFFFF), 0x3F800000)
    m = lax.bitcast_convert_type(mbits, jnp.float32)
    big = m > _SQRT2
    m = jnp.where(big, m * 0.5, m)
    e = jnp.where(big, e + 1, e)
    t = (m - 1.0) / (m + 1.0)
    t2 = t * t
    p = t * (2.0 + t2 * (2.0 / 3.0 + t2 * (0.4 + t2 * (2.0 / 7.0))))
    return e.astype(jnp.float32) * _LN2 + p


def _xlane(x, op):
    dnums = lax.GatherDimensionNumbers(
        offset_dims=(), collapsed_slice_dims=(0,), start_index_map=(0,))
    for step in (1, 2, 4, 8):
        perm = lax.bitwise_xor(lax.iota(jnp.int32, _LANES), step)
        shuf = lax.gather(x, perm.reshape(_LANES, 1), dnums, (1,),
                          mode=lax.GatherScatterMode.PROMISE_IN_BOUNDS)
        x = op(x, shuf)
    return x


def _sc_body(wf_hbm, base_hbm, out_hbm, idx_v, idxb, rowb, sem):
    wid = lax.axis_index("s") * _NUM_CORES + lax.axis_index("c")
    pltpu.sync_copy(base_hbm, idx_v)
    e = idx_v[...][0]
    row0 = wid * _ROWS_PER_W

    @pl.when(row0 < _SEQ_LEN)
    def _():
        lane = lax.iota(jnp.int32, _LANES)
        lanepat = (lax.shift_right_logical(lane, 3) * 8192
                   + lax.bitwise_and(lane, 7) * 128)
        ebase = (lax.shift_right_logical(e, 7) * 1024
                 + lax.bitwise_and(e, 127))

        for r in range(_ROWS_PER_W):
            rowbase = (row0 + r) * _ROW_STRIDE + ebase
            for g in range(_NV):
                iv = jnp.full((_LANES,), rowbase + g * _G_STRIDE,
                              jnp.int32) + lanepat
                if g == _NV - 2:
                    iv = jnp.where(lane < 8, iv,
                                   jnp.full((_LANES,), rowbase, jnp.int32))
                elif g == _NV - 1:
                    iv = jnp.full((_LANES,), rowbase, jnp.int32)
                idxb[pl.ds(r * _VPAD + g * _LANES, _LANES)] = iv
        copies = []
        for r in range(_ROWS_PER_W):
            for ch in range(_NCH):
                copies.append(pltpu.async_copy(
                    wf_hbm.at[idxb.at[pl.ds(r * _VPAD + ch * _CHUNK,
                                            _CHUNK)]],
                    rowb.at[pl.ds(r * _VPAD + ch * _CHUNK, _CHUNK)],
                    sem))
        for cp in copies:
            cp.wait()

        ninf = jnp.full((_LANES,), -jnp.inf, jnp.float32)
        for r in range(_ROWS_PER_W):
            b0 = r * _VPAD
            v = rowb[pl.ds(b0 + 992, _LANES)]
            rowb[pl.ds(b0 + 992, _LANES)] = jnp.where(lane < 8, v, ninf)
            rowb[pl.ds(b0 + 1008, _LANES)] = ninf

        for r in range(_ROWS_PER_W):
            b0 = r * _VPAD
            m = rowb[pl.ds(b0, _LANES)]
            for k in range(1, _NV):
                m = jnp.maximum(m, rowb[pl.ds(b0 + k * _LANES, _LANES)])
            mb = _xlane(m, jnp.maximum)
            s = jnp.zeros((_LANES,), jnp.float32)
            for k in range(_NV):
                s = s + jnp.exp(rowb[pl.ds(b0 + k * _LANES, _LANES)] - mb)
            off = mb + _vlog(_xlane(s, jnp.add))
            for k in range(_NV):
                rowb[pl.ds(b0 + k * _LANES, _LANES)] = (
                    rowb[pl.ds(b0 + k * _LANES, _LANES)] - off)
            pltpu.sync_copy(
                rowb.at[pl.ds(b0, _VOCAB)],
                out_hbm.at[pl.ds((row0 + r) * _VOCAB, _VOCAB)])


_sc_kernel = functools.partial(
    pl.kernel,
    mesh=plsc.VectorSubcoreMesh(core_axis_name="c", subcore_axis_name="s"),
    out_type=jax.ShapeDtypeStruct((_SEQ_LEN * _VOCAB,), jnp.float32),
    scratch_types=[
        pltpu.VMEM((_LANES,), jnp.int32),
        pltpu.VMEM((_ROWS_PER_W * _VPAD,), jnp.int32),
        pltpu.VMEM((_ROWS_PER_W * _VPAD,), jnp.float32),
        pltpu.SemaphoreType.DMA,
    ],
)(_sc_body)


def kernel(weights, example_idx):
    wf = (weights.transpose(1, 2, 0)
          .reshape(_SEQ_LEN * _VOCAB // 8, 8, _NUM_EXAMPLES // 128, 128)
          .swapaxes(1, 2)
          .reshape(_SEQ_LEN * _VOCAB * _NUM_EXAMPLES))
    base = jnp.broadcast_to(jnp.asarray(example_idx, jnp.int32), (_LANES,))
    return _sc_kernel(wf, base).reshape(_SEQ_LEN, _VOCAB)


# speedup vs baseline: 18.5784x; 1.0214x over previous
import functools

import jax
import jax.numpy as jnp
from jax import lax
from jax.experimental import pallas as pl
from jax.experimental.pallas import tpu as pltpu
from jax.experimental.pallas import tpu_sc as plsc

_NUM_EXAMPLES = 1024
_SEQ_LEN = 50
_VOCAB = 1000

_LANES = 16
_NUM_CORES = 2
_ROWS_PER_W = 2
_VPAD = 1024
_NV = _VPAD // _LANES
_CHUNK = 1024
_NCH = _VPAD // _CHUNK
_ROW_STRIDE = _SEQ_LEN * _VOCAB * _NUM_EXAMPLES // _SEQ_LEN
_G_STRIDE = 16384

_LN2 = 0.6931471805599453
_SQRT2 = 1.4142135623730951


def _vlog(x):
    bits = lax.bitcast_convert_type(x, jnp.int32)
    e = lax.shift_right_logical(bits, 23) - 127
    mbits = lax.bitwise_or(lax.bitwise_and(bits, 0x007FFFFF), 0x3F800000)
    m = lax.bitcast_convert_type(mbits, jnp.float32)
    big = m > _SQRT2
    m = jnp.where(big, m * 0.5, m)
    e = jnp.where(big, e + 1, e)
    t = (m - 1.0) / (m + 1.0)
    t2 = t * t
    p = t * (2.0 + t2 * (2.0 / 3.0 + t2 * (0.4 + t2 * (2.0 / 7.0))))
    return e.astype(jnp.float32) * _LN2 + p


def _xlane(x, op):
    dnums = lax.GatherDimensionNumbers(
        offset_dims=(), collapsed_slice_dims=(0,), start_index_map=(0,))
    for step in (1, 2, 4, 8):
        perm = lax.bitwise_xor(lax.iota(jnp.int32, _LANES), step)
        shuf = lax.gather(x, perm.reshape(_LANES, 1), dnums, (1,),
                          mode=lax.GatherScatterMode.PROMISE_IN_BOUNDS)
        x = op(x, shuf)
    return x


def _sc_body(wf_hbm, base_hbm, out_hbm, idx_v, idxb, rowb, sem):
    wid = lax.axis_index("s") * _NUM_CORES + lax.axis_index("c")
    pltpu.sync_copy(base_hbm, idx_v)
    e = idx_v[...][0]
    row0 = wid * _ROWS_PER_W

    @pl.when(row0 < _SEQ_LEN)
    def _():
        lane = lax.iota(jnp.int32, _LANES)
        lanepat = (lax.shift_right_logical(lane, 3) * 8192
                   + lax.bitwise_and(lane, 7) * 128)
        ebase = (lax.shift_right_logical(e, 7) * 1024
                 + lax.bitwise_and(e, 127))

        for r in range(_ROWS_PER_W):
            rowbase = (row0 + r) * _ROW_STRIDE + ebase
            for g in range(_NV):
                iv = jnp.full((_LANES,), rowbase + g * _G_STRIDE,
                              jnp.int32) + lanepat
                if g == _NV - 2:
                    iv = jnp.where(lane < 8, iv,
                                   jnp.full((_LANES,), rowbase, jnp.int32))
                elif g == _NV - 1:
                    iv = jnp.full((_LANES,), rowbase, jnp.int32)
                idxb[pl.ds(r * _VPAD + g * _LANES, _LANES)] = iv
        copies = []
        for r in range(_ROWS_PER_W):
            for ch in range(_NCH):
                copies.append(pltpu.async_copy(
                    wf_hbm.at[idxb.at[pl.ds(r * _VPAD + ch * _CHUNK,
                                            _CHUNK)]],
                    rowb.at[pl.ds(r * _VPAD + ch * _CHUNK, _CHUNK)],
                    sem))
        for cp in copies:
            cp.wait()

        ninf = jnp.full((_LANES,), -jnp.inf, jnp.float32)
        for r in range(_ROWS_PER_W):
            b0 = r * _VPAD
            v = rowb[pl.ds(b0 + 992, _LANES)]
            rowb[pl.ds(b0 + 992, _LANES)] = jnp.where(lane < 8, v, ninf)
            rowb[pl.ds(b0 + 1008, _LANES)] = ninf

        for r in range(_ROWS_PER_W):
            b0 = r * _VPAD
            m = rowb[pl.ds(b0, _LANES)]
            for k in range(1, _NV):
                m = jnp.maximum(m, rowb[pl.ds(b0 + k * _LANES, _LANES)])
            mb = _xlane(m, jnp.maximum)
            s = jnp.zeros((_LANES,), jnp.float32)
            for k in range(_NV):
                s = s + jnp.exp(rowb[pl.ds(b0 + k * _LANES, _LANES)] - mb)
            off = mb + _vlog(_xlane(s, jnp.add))
            for k in range(_NV):
                rowb[pl.ds(b0 + k * _LANES, _LANES)] = (
                    rowb[pl.ds(b0 + k * _LANES, _LANES)] - off)
            pltpu.sync_copy(
                rowb.at[pl.ds(b0, _VOCAB)],
                out_hbm.at[pl.ds((row0 + r) * _VOCAB, _VOCAB)])


_sc_kernel = functools.partial(
    pl.kernel,
    mesh=plsc.VectorSubcoreMesh(core_axis_name="c", subcore_axis_name="s"),
    out_type=jax.ShapeDtypeStruct((_SEQ_LEN * _VOCAB,), jnp.float32),
    scratch_types=[
        pltpu.VMEM((_LANES,), jnp.int32),
        pltpu.VMEM((_ROWS_PER_W * _VPAD,), jnp.int32),
        pltpu.VMEM((_ROWS_PER_W * _VPAD,), jnp.float32),
        pltpu.SemaphoreType.DMA,
    ],
)(_sc_body)


def kernel(weights, example_idx):
    wf = (weights.transpose(1, 2, 0)
          .reshape(_SEQ_LEN * _VOCAB // 8, 8, _NUM_EXAMPLES // 128, 128)
          .swapaxes(1, 2)
          .reshape(_SEQ_LEN * _VOCAB * _NUM_EXAMPLES))
    base = jnp.broadcast_to(jnp.asarray(example_idx, jnp.int32), (_LANES,))
    return _sc_kernel(wf, base).reshape(_SEQ_LEN, _VOCAB)
